# Initial kernel scaffold; baseline (speedup 1.0000x reference)
import jax
import jax.numpy as jnp
from jax.experimental import pallas as pl


def kernel(preds, labels, val_preds):
    raise NotImplementedError("write your pallas kernel here")


# speedup vs baseline: 1.0399x; 1.0399x over previous
import functools

import jax
import jax.numpy as jnp
from jax import lax
from jax.experimental import pallas as pl
from jax.experimental.pallas import tpu as pltpu
from jax.experimental.pallas import tpu_sc as plsc

_GAMMA = 0.9
_N = 16384
_C = 1000
_W = 512
_W1 = _C - _W
_CHUNK = 64
_NCHUNK = _N // (16 * _CHUNK)
_UNITS = _C // 8


def _zero16():
    return jnp.zeros((16,), jnp.float32)


def _body(preds_hbm, labels_hbm, vp_hbm, out_hbm,
          buf, idx_v, ones_v, z8, s8, v8, counts_v, acc_sp, cnt_sp):
    cid = lax.axis_index("c")
    sid = lax.axis_index("s")
    is0 = cid == 0

    def _zrow(r, _):
        def _zc(c, _):
            z8[r, pl.ds(c * 16, 16)] = _zero16()
            return 0
        return lax.fori_loop(0, _W // 16, _zc, 0)
    lax.fori_loop(0, 8, _zrow, 0)

    def _obuf(i, _):
        ones_v[pl.ds(i * 16, 16)] = _zero16() + 1.0
        buf[i, pl.ds(_W1, 16)] = _zero16()
        buf[i, pl.ds(_W1 + 8, 16)] = _zero16()
        return 0
    lax.fori_loop(0, _CHUNK // 16, _obuf, 0)

    def _obuf2(i, _):
        buf[i, pl.ds(_W1, 16)] = _zero16()
        buf[i, pl.ds(_W1 + 8, 16)] = _zero16()
        return 0
    lax.fori_loop(_CHUNK // 16, _CHUNK, _obuf2, 0)

    def _zunit(j, _):
        u = sid + j * 16

        @pl.when(u < _UNITS)
        def _():
            pltpu.sync_copy(z8, acc_sp.at[pl.ds(u * 8, 8)])
            pltpu.sync_copy(z8.at[0, pl.ds(0, 8)], cnt_sp.at[pl.ds(u * 8, 8)])
        return 0
    lax.fori_loop(0, (_UNITS + 15) // 16, _zunit, 0)

    plsc.subcore_barrier()

    base = sid * (_N // 16)

    def _chunk(k, _):
        r0 = base + k * _CHUNK
        pltpu.sync_copy(labels_hbm.at[pl.ds(r0, _CHUNK)], idx_v)

        @pl.when(is0)
        def _():
            pltpu.sync_copy(preds_hbm.at[pl.ds(r0, _CHUNK), pl.ds(0, _W)], buf)

        @pl.when(jnp.logical_not(is0))
        def _():
            pltpu.sync_copy(preds_hbm.at[pl.ds(r0, _CHUNK), pl.ds(_W, _W1)],
                            buf.at[:, pl.ds(0, _W1)])

        pltpu.sync_copy(buf, acc_sp.at[idx_v], add=True)
        pltpu.sync_copy(ones_v, cnt_sp.at[idx_v], add=True)
        return 0
    lax.fori_loop(0, _NCHUNK, _chunk, 0)

    plsc.subcore_barrier()

    pltpu.sync_copy(cnt_sp, counts_v)
    gamma = jnp.float32(_GAMMA)
    one_m_gamma = jnp.float32(1.0 - _GAMMA)

    def _unit(j, _):
        u = sid + j * 16

        @pl.when(u < _UNITS)
        def _():
            r0 = u * 8
            pltpu.sync_copy(acc_sp.at[pl.ds(r0, 8)], s8)

            @pl.when(is0)
            def _():
                pltpu.sync_copy(vp_hbm.at[pl.ds(r0, 8), pl.ds(0, _W)], v8)

            @pl.when(jnp.logical_not(is0))
            def _():
                pltpu.sync_copy(vp_hbm.at[pl.ds(r0, 8), pl.ds(_W, _W1)],
                                v8.at[:, pl.ds(0, _W1)])

            def _row(r, _):
                ridx = jnp.zeros((16,), jnp.int32) + (r0 + r)
                cnt = plsc.load_gather(counts_v, [ridx])
                has = cnt > 0.0
                a = jnp.where(has, one_m_gamma / jnp.where(has, cnt, 1.0), 0.0)
                b = jnp.where(has, gamma, 1.0)

                def _col(c, _):
                    s = s8[r, pl.ds(c * 16, 16)]
                    v = v8[r, pl.ds(c * 16, 16)]
                    s8[r, pl.ds(c * 16, 16)] = s * a + v * b
                    return 0
                return lax.fori_loop(0, _W // 16, _col, 0)
            lax.fori_loop(0, 8, _row, 0)

            @pl.when(is0)
            def _():
                pltpu.sync_copy(s8, out_hbm.at[pl.ds(r0, 8), pl.ds(0, _W)])

            @pl.when(jnp.logical_not(is0))
            def _():
                pltpu.sync_copy(s8.at[:, pl.ds(0, _W1)],
                                out_hbm.at[pl.ds(r0, 8), pl.ds(_W, _W1)])
        return 0
    lax.fori_loop(0, (_UNITS + 15) // 16, _unit, 0)


@functools.partial(
    pl.kernel,
    out_type=jax.ShapeDtypeStruct((_C, _C), jnp.float32),
    mesh=plsc.VectorSubcoreMesh(core_axis_name="c", subcore_axis_name="s"),
    compiler_params=pltpu.CompilerParams(
        use_tc_tiling_on_sc=False, needs_layout_passes=False),
    scratch_types=[
        pltpu.VMEM((_CHUNK, _W), jnp.float32),
        pltpu.VMEM((_CHUNK,), jnp.int32),
        pltpu.VMEM((_CHUNK,), jnp.float32),
        pltpu.VMEM((8, _W), jnp.float32),
        pltpu.VMEM((8, _W), jnp.float32),
        pltpu.VMEM((8, _W), jnp.float32),
        pltpu.VMEM((_C,), jnp.float32),
        pltpu.VMEM_SHARED((_C, _W), jnp.float32),
        pltpu.VMEM_SHARED((_C,), jnp.float32),
    ],
)
def _val_scores(preds_hbm, labels_hbm, vp_hbm, out_hbm, *scratch):
    _body(preds_hbm, labels_hbm, vp_hbm, out_hbm, *scratch)


def kernel(preds, labels, val_preds):
    return _val_scores(preds, labels, val_preds)


# speedup vs baseline: 1.1491x; 1.1050x over previous
import functools

import jax
import jax.numpy as jnp
from jax import lax
from jax.experimental import pallas as pl
from jax.experimental.pallas import tpu as pltpu
from jax.experimental.pallas import tpu_sc as plsc

_GAMMA = 0.9
_N = 16384
_C = 1000
_W = 512
_CHUNK = 64
_NCHUNK = _N // (16 * _CHUNK)
_UNITS = _C // 8


def _zero16():
    return jnp.zeros((16,), jnp.float32)


def _body(preds_hbm, labels_hbm, vp_hbm, out_hbm,
          buf2, idx2, ones_v, z8, s8, v8, counts_v, sems, acc_sp, cnt_sp):
    cid = lax.axis_index("c")
    sid = lax.axis_index("s")
    col0 = cid * (_C - _W)

    def _zrow(r, _):
        def _zc(c, _):
            z8[r, pl.ds(c * 16, 16)] = _zero16()
            return 0
        return lax.fori_loop(0, _W // 16, _zc, 0)
    lax.fori_loop(0, 8, _zrow, 0)

    def _obuf(i, _):
        ones_v[pl.ds(i * 16, 16)] = _zero16() + 1.0
        return 0
    lax.fori_loop(0, _CHUNK // 16, _obuf, 0)

    def _zunit(j, _):
        u = sid + j * 16

        @pl.when(u < _UNITS)
        def _():
            pltpu.sync_copy(z8, acc_sp.at[pl.ds(u * 8, 8)])
            pltpu.sync_copy(z8.at[0, pl.ds(0, 8)], cnt_sp.at[pl.ds(u * 8, 8)])
        return 0
    lax.fori_loop(0, (_UNITS + 15) // 16, _zunit, 0)

    plsc.subcore_barrier()

    base = sid * (_N // 16)

    def _start_fetch(k, slot):
        r0 = base + k * _CHUNK
        pltpu.async_copy(labels_hbm.at[pl.ds(r0, _CHUNK)], idx2.at[slot],
                         sems.at[slot])
        pltpu.async_copy(preds_hbm.at[pl.ds(r0, _CHUNK), pl.ds(col0, _W)],
                         buf2.at[slot], sems.at[slot])

    def _wait_fetch(slot):
        pltpu.make_async_copy(labels_hbm.at[pl.ds(0, _CHUNK)], idx2.at[slot],
                              sems.at[slot]).wait()
        pltpu.make_async_copy(preds_hbm.at[pl.ds(0, _CHUNK), pl.ds(0, _W)],
                              buf2.at[slot], sems.at[slot]).wait()

    _start_fetch(0, 0)
    for k in range(_NCHUNK):
        slot = k % 2
        _wait_fetch(slot)
        if k + 1 < _NCHUNK:
            _start_fetch(k + 1, 1 - slot)
        pltpu.sync_copy(buf2.at[slot], acc_sp.at[idx2.at[slot]], add=True)
        pltpu.sync_copy(ones_v, cnt_sp.at[idx2.at[slot]], add=True)

    plsc.subcore_barrier()

    pltpu.sync_copy(cnt_sp, counts_v)
    gamma = jnp.float32(_GAMMA)
    one_m_gamma = jnp.float32(1.0 - _GAMMA)

    def _unit(j, _):
        u = sid + j * 16

        @pl.when(u < _UNITS)
        def _():
            r0 = u * 8
            pltpu.sync_copy(acc_sp.at[pl.ds(r0, 8)], s8)
            pltpu.sync_copy(vp_hbm.at[pl.ds(r0, 8), pl.ds(col0, _W)], v8)

            def _row(r, _):
                ridx = jnp.zeros((16,), jnp.int32) + (r0 + r)
                cnt = plsc.load_gather(counts_v, [ridx])
                has = cnt > 0.0
                a = jnp.where(has, one_m_gamma / jnp.where(has, cnt, 1.0), 0.0)
                b = jnp.where(has, gamma, 1.0)

                def _col(c, _):
                    s = s8[r, pl.ds(c * 16, 16)]
                    v = v8[r, pl.ds(c * 16, 16)]
                    s8[r, pl.ds(c * 16, 16)] = s * a + v * b
                    return 0
                return lax.fori_loop(0, _W // 16, _col, 0)
            lax.fori_loop(0, 8, _row, 0)

            pltpu.sync_copy(s8, out_hbm.at[pl.ds(r0, 8), pl.ds(col0, _W)])
        return 0
    lax.fori_loop(0, (_UNITS + 15) // 16, _unit, 0)


@functools.partial(
    pl.kernel,
    out_type=jax.ShapeDtypeStruct((_C, _C), jnp.float32),
    mesh=plsc.VectorSubcoreMesh(core_axis_name="c", subcore_axis_name="s"),
    compiler_params=pltpu.CompilerParams(
        use_tc_tiling_on_sc=False, needs_layout_passes=False),
    scratch_types=[
        pltpu.VMEM((2, _CHUNK, _W), jnp.float32),
        pltpu.VMEM((2, _CHUNK), jnp.int32),
        pltpu.VMEM((_CHUNK,), jnp.float32),
        pltpu.VMEM((8, _W), jnp.float32),
        pltpu.VMEM((8, _W), jnp.float32),
        pltpu.VMEM((8, _W), jnp.float32),
        pltpu.VMEM((_C,), jnp.float32),
        pltpu.SemaphoreType.DMA((2,)),
        pltpu.VMEM_SHARED((_C, _W), jnp.float32),
        pltpu.VMEM_SHARED((_C,), jnp.float32),
    ],
)
def _val_scores(preds_hbm, labels_hbm, vp_hbm, out_hbm, *scratch):
    _body(preds_hbm, labels_hbm, vp_hbm, out_hbm, *scratch)


def kernel(preds, labels, val_preds):
    return _val_scores(preds, labels, val_preds)


# speedup vs baseline: 1.2301x; 1.0705x over previous
import functools

import jax
import jax.numpy as jnp
from jax import lax
from jax.experimental import pallas as pl
from jax.experimental.pallas import tpu as pltpu
from jax.experimental.pallas import tpu_sc as plsc

_GAMMA = 0.9
_N = 16384
_C = 1000
_W = 512
_CHUNK = 64
_NCHUNK = _N // (16 * _CHUNK)
_UNITS = _C // 8


def _zero16():
    return jnp.zeros((16,), jnp.float32)


def _body(preds_hbm, labels_hbm, sums_hbm, cnt_hbm,
          buf2, idx2, ones_v, z8, sems, acc_sp, cnt_sp):
    cid = lax.axis_index("c")
    sid = lax.axis_index("s")
    col0 = cid * (_C - _W)

    def _zrow(r, _):
        def _zc(c, _):
            z8[r, pl.ds(c * 16, 16)] = _zero16()
            return 0
        return lax.fori_loop(0, _W // 16, _zc, 0)
    lax.fori_loop(0, 8, _zrow, 0)

    def _obuf(i, _):
        ones_v[pl.ds(i * 16, 16)] = _zero16() + 1.0
        return 0
    lax.fori_loop(0, _CHUNK // 16, _obuf, 0)

    def _zunit(j, _):
        u = sid + j * 16

        @pl.when(u < _UNITS)
        def _():
            pltpu.sync_copy(z8, acc_sp.at[pl.ds(u * 8, 8)])
            pltpu.sync_copy(z8.at[0, pl.ds(0, 8)], cnt_sp.at[pl.ds(u * 8, 8)])
        return 0
    lax.fori_loop(0, (_UNITS + 15) // 16, _zunit, 0)

    plsc.subcore_barrier()

    base = sid * (_N // 16)

    def _start_fetch(k, slot):
        r0 = base + k * _CHUNK
        pltpu.async_copy(labels_hbm.at[pl.ds(r0, _CHUNK)], idx2.at[slot],
                         sems.at[slot])
        pltpu.async_copy(preds_hbm.at[pl.ds(r0, _CHUNK), pl.ds(col0, _W)],
                         buf2.at[slot], sems.at[slot])

    def _wait_fetch(slot):
        pltpu.make_async_copy(labels_hbm.at[pl.ds(0, _CHUNK)], idx2.at[slot],
                              sems.at[slot]).wait()
        pltpu.make_async_copy(preds_hbm.at[pl.ds(0, _CHUNK), pl.ds(0, _W)],
                              buf2.at[slot], sems.at[slot]).wait()

    _start_fetch(0, 0)
    for k in range(_NCHUNK):
        slot = k % 2
        _wait_fetch(slot)
        if k + 1 < _NCHUNK:
            _start_fetch(k + 1, 1 - slot)
        pltpu.sync_copy(buf2.at[slot], acc_sp.at[idx2.at[slot]], add=True)
        pltpu.sync_copy(ones_v, cnt_sp.at[idx2.at[slot]], add=True)

    plsc.subcore_barrier()

    def _wunit(j, _):
        u = sid + j * 16

        @pl.when(u < _UNITS)
        def _():
            pltpu.sync_copy(acc_sp.at[pl.ds(u * 8, 8)],
                            sums_hbm.at[pl.ds(u * 8, 8), pl.ds(col0, _W)])
        return 0
    lax.fori_loop(0, (_UNITS + 15) // 16, _wunit, 0)

    @pl.when(jnp.logical_and(cid == 0, sid < 8))
    def _():
        pltpu.sync_copy(cnt_sp.at[pl.ds(sid * 128, 128)],
                        cnt_hbm.at[pl.ds(sid * 128, 128)])


@functools.partial(
    pl.kernel,
    out_type=(
        jax.ShapeDtypeStruct((_C, _C), jnp.float32),
        jax.ShapeDtypeStruct((1024,), jnp.float32),
    ),
    mesh=plsc.VectorSubcoreMesh(core_axis_name="c", subcore_axis_name="s"),
    compiler_params=pltpu.CompilerParams(
        use_tc_tiling_on_sc=False, needs_layout_passes=False),
    scratch_types=[
        pltpu.VMEM((2, _CHUNK, _W), jnp.float32),
        pltpu.VMEM((2, _CHUNK), jnp.int32),
        pltpu.VMEM((_CHUNK,), jnp.float32),
        pltpu.VMEM((8, _W), jnp.float32),
        pltpu.SemaphoreType.DMA((2,)),
        pltpu.VMEM_SHARED((_C, _W), jnp.float32),
        pltpu.VMEM_SHARED((1024,), jnp.float32),
    ],
)
def _segment_sums(preds_hbm, labels_hbm, sums_hbm, cnt_hbm, *scratch):
    _body(preds_hbm, labels_hbm, sums_hbm, cnt_hbm, *scratch)


_BLK = 200


def _ema_body(s_ref, c_ref, vp_ref, out_ref):
    cnt = c_ref[...]
    has = cnt > 0.0
    a = jnp.where(has, (1.0 - _GAMMA) / jnp.where(has, cnt, 1.0), 0.0)
    b = jnp.where(has, jnp.float32(_GAMMA), 1.0)
    out_ref[...] = s_ref[...] * a + vp_ref[...] * b


_ema = pl.pallas_call(
    _ema_body,
    grid=(_C // _BLK,),
    in_specs=[
        pl.BlockSpec((_BLK, _C), lambda i: (i, 0)),
        pl.BlockSpec((_BLK, 1), lambda i: (i, 0)),
        pl.BlockSpec((_BLK, _C), lambda i: (i, 0)),
    ],
    out_specs=pl.BlockSpec((_BLK, _C), lambda i: (i, 0)),
    out_shape=jax.ShapeDtypeStruct((_C, _C), jnp.float32),
)


def kernel(preds, labels, val_preds):
    sums, cnt = _segment_sums(preds, labels)
    return _ema(sums, cnt[:_C, None], val_preds)
